# Initial kernel scaffold; baseline (speedup 1.0000x reference)
import jax
import jax.numpy as jnp
from jax.experimental import pallas as pl


def kernel(x, index):
    raise NotImplementedError("write your pallas kernel here")


# speedup vs baseline: 1.1103x; 1.1103x over previous
import functools

import numpy as np
import jax
import jax.numpy as jnp
from jax import lax
from jax.experimental import pallas as pl
from jax.experimental.pallas import tpu as pltpu
from jax.experimental.pallas import tpu_sc as plsc

D_MODEL = 768
MAX_LEN = 5000
BATCH = 4
SEQ = 4096
ROWS = BATCH * SEQ
XSCALE = float(np.sqrt(float(D_MODEL)))

NC = 2
NS = 16
NW = NC * NS
RPW = ROWS // NW
CH = 64
NCHUNK = RPW // CH
LANES = 16
DV = D_MODEL // LANES


def _pe_table_np():
    position = np.arange(MAX_LEN, dtype=np.float32)[:, None]
    div_term = np.exp(
        np.arange(0, D_MODEL, 2, dtype=np.float32) * (-np.log(10000.0) / D_MODEL)
    )
    pe = np.zeros((MAX_LEN, D_MODEL), dtype=np.float32)
    pe[:, 0::2] = np.sin(position * div_term)
    pe[:, 1::2] = np.cos(position * div_term)
    return pe


_PE_NP = _pe_table_np()


@functools.partial(
    pl.kernel,
    mesh=plsc.VectorSubcoreMesh(core_axis_name="c", subcore_axis_name="s"),
    out_type=jax.ShapeDtypeStruct((ROWS, D_MODEL), jnp.float32),
    scratch_types=[
        pltpu.VMEM((CH,), jnp.int32),
        pltpu.VMEM((CH, D_MODEL), jnp.float32),
        pltpu.VMEM((CH, D_MODEL), jnp.float32),
        pltpu.SemaphoreType.DMA,
    ],
)
def _sc_encode(x_hbm, idx_hbm, pe_hbm, out_hbm, idx_v, x_v, pe_v, sem):
    cid = lax.axis_index("c")
    sid = lax.axis_index("s")
    wid = sid * NC + cid
    base = wid * RPW

    def chunk_body(c, carry):
        rbase = base + c * CH
        pltpu.sync_copy(idx_hbm.at[pl.ds(rbase, CH)], idx_v)
        gather = pltpu.async_copy(pe_hbm.at[idx_v], pe_v, sem)
        pltpu.sync_copy(x_hbm.at[pl.ds(rbase, CH)], x_v)
        gather.wait()

        def row_body(r, rcarry):
            for j in range(DV):
                sl = pl.ds(j * LANES, LANES)
                pe_v[r, sl] = x_v[r, sl] * XSCALE + pe_v[r, sl]
            return rcarry

        lax.fori_loop(0, CH, row_body, 0)
        pltpu.sync_copy(pe_v, out_hbm.at[pl.ds(rbase, CH)])
        return carry

    lax.fori_loop(0, NCHUNK, chunk_body, 0)


def kernel(x, index):
    pe = jnp.asarray(_PE_NP)
    xf = x.reshape(ROWS, D_MODEL)
    idxf = index.reshape(ROWS).astype(jnp.int32)
    out = _sc_encode(xf, idxf, pe)
    return out.reshape(x.shape)


# speedup vs baseline: 1.5152x; 1.3646x over previous
import functools

import numpy as np
import jax
import jax.numpy as jnp
from jax import lax
from jax.experimental import pallas as pl
from jax.experimental.pallas import tpu as pltpu
from jax.experimental.pallas import tpu_sc as plsc

D_MODEL = 768
MAX_LEN = 5000
BATCH = 4
SEQ = 4096
ROWS = BATCH * SEQ
XSCALE = float(np.sqrt(float(D_MODEL)))

NC = 2
NS = 16
NW = NC * NS
RPW = ROWS // NW
CH = 32
NCHUNK = RPW // CH
NPAIR = NCHUNK // 2
LANES = 16
DV = D_MODEL // LANES


def _pe_table_np():
    position = np.arange(MAX_LEN, dtype=np.float32)[:, None]
    div_term = np.exp(
        np.arange(0, D_MODEL, 2, dtype=np.float32) * (-np.log(10000.0) / D_MODEL)
    )
    pe = np.zeros((MAX_LEN, D_MODEL), dtype=np.float32)
    pe[:, 0::2] = np.sin(position * div_term)
    pe[:, 1::2] = np.cos(position * div_term)
    return pe


_PE_NP = _pe_table_np()


@functools.partial(
    pl.kernel,
    mesh=plsc.VectorSubcoreMesh(core_axis_name="c", subcore_axis_name="s"),
    out_type=jax.ShapeDtypeStruct((ROWS, D_MODEL), jnp.float32),
    scratch_types=[
        pltpu.VMEM((RPW,), jnp.int32),
        pltpu.VMEM((CH, D_MODEL), jnp.float32),
        pltpu.VMEM((CH, D_MODEL), jnp.float32),
        pltpu.VMEM((CH, D_MODEL), jnp.float32),
        pltpu.VMEM((CH, D_MODEL), jnp.float32),
        pltpu.SemaphoreType.DMA,
        pltpu.SemaphoreType.DMA,
        pltpu.SemaphoreType.DMA,
        pltpu.SemaphoreType.DMA,
        pltpu.SemaphoreType.DMA,
        pltpu.SemaphoreType.DMA,
    ],
)
def _sc_encode(x_hbm, idx_hbm, pe_hbm, out_hbm,
               idx_v, x0, x1, pe0, pe1, g0, g1, xs0, xs1, s0, s1):
    cid = lax.axis_index("c")
    sid = lax.axis_index("s")
    wid = sid * NC + cid
    base = wid * RPW

    pltpu.sync_copy(idx_hbm.at[pl.ds(base, RPW)], idx_v)

    def issue_loads(c, pe_v, x_v, gsem, xsem):
        pltpu.async_copy(pe_hbm.at[idx_v.at[pl.ds(c * CH, CH)]], pe_v, gsem)
        pltpu.async_copy(x_hbm.at[pl.ds(base + c * CH, CH)], x_v, xsem)

    def wait_loads(c, pe_v, x_v, gsem, xsem):
        pltpu.make_async_copy(
            pe_hbm.at[idx_v.at[pl.ds(c * CH, CH)]], pe_v, gsem).wait()
        pltpu.make_async_copy(
            x_hbm.at[pl.ds(base + c * CH, CH)], x_v, xsem).wait()

    def wait_store(c, pe_v, ssem):
        pltpu.make_async_copy(
            pe_v, out_hbm.at[pl.ds(base + c * CH, CH)], ssem).wait()

    def compute(pe_v, x_v):
        def row_body(r, rcarry):
            for j in range(DV):
                sl = pl.ds(j * LANES, LANES)
                plsc.addupdate(pe_v.at[r, sl], x_v[r, sl] * XSCALE)
            return rcarry

        lax.fori_loop(0, CH, row_body, 0)

    issue_loads(0, pe0, x0, g0, xs0)
    issue_loads(1, pe1, x1, g1, xs1)

    def pair_body(p, carry):
        c0 = 2 * p
        c1 = 2 * p + 1

        wait_loads(c0, pe0, x0, g0, xs0)
        compute(pe0, x0)
        pltpu.async_copy(pe0, out_hbm.at[pl.ds(base + c0 * CH, CH)], s0)

        @pl.when(p < NPAIR - 1)
        def _():
            wait_store(c0, pe0, s0)
            issue_loads(c0 + 2, pe0, x0, g0, xs0)

        wait_loads(c1, pe1, x1, g1, xs1)
        compute(pe1, x1)
        pltpu.async_copy(pe1, out_hbm.at[pl.ds(base + c1 * CH, CH)], s1)

        @pl.when(p < NPAIR - 1)
        def _():
            wait_store(c1, pe1, s1)
            issue_loads(c1 + 2, pe1, x1, g1, xs1)

        return carry

    lax.fori_loop(0, NPAIR, pair_body, 0)

    wait_store(NCHUNK - 2, pe0, s0)
    wait_store(NCHUNK - 1, pe1, s1)


def kernel(x, index):
    pe = jnp.asarray(_PE_NP)
    xf = x.reshape(ROWS, D_MODEL)
    idxf = index.reshape(ROWS).astype(jnp.int32)
    out = _sc_encode(xf, idxf, pe)
    return out.reshape(x.shape)
